# baseline (speedup 1.0000x reference)
import functools

import jax
import jax.numpy as jnp
from jax import lax
from jax.experimental import pallas as pl
from jax.experimental.pallas import tpu as pltpu
from jax.experimental.pallas import tpu_sc as plsc

NUM_ENT = 1000000
NUM_REL = 1000
DIM = 32
B = 16384
PACK = 4

_NC, _NS = 2, 16
_NW = _NC * _NS
_BPW = B // _NW
_BLK = 16
_CH = 128
_NCH = _BPW // _CH
_BPC = _CH // _BLK


def _soft_rsqrt(x):
    i = lax.bitcast_convert_type(x, jnp.int32)
    i = jnp.int32(0x5F3759DF) - lax.shift_right_arithmetic(i, jnp.int32(1))
    y = lax.bitcast_convert_type(i, jnp.float32)
    for _ in range(3):
        y = y * (1.5 - 0.5 * x * y * y)
    return y


def _soft_sqrt(x):
    return x * _soft_rsqrt(x)


def _soft_log(x):
    i = lax.bitcast_convert_type(x, jnp.int32)
    e = lax.shift_right_arithmetic(i, jnp.int32(23)) - jnp.int32(127)
    mbits = lax.bitwise_or(lax.bitwise_and(i, jnp.int32(0x007FFFFF)),
                           jnp.int32(0x3F800000))
    m = lax.bitcast_convert_type(mbits, jnp.float32)
    big = m > 1.4142135381698608
    m = jnp.where(big, m * 0.5, m)
    e = jnp.where(big, e + jnp.int32(1), e)
    ef = e.astype(jnp.float32)
    t = (m - 1.0) / (m + 1.0)
    t2 = t * t
    p = 1.0 + t2 * (0.3333333333 + t2 * (0.2 + t2 * (0.14285714 + t2 * 0.11111111)))
    return ef * 0.6931471805599453 + 2.0 * t * p


def _soft_artanh(x):
    return 0.5 * _soft_log((1.0 + x) / (1.0 - x))


def _soft_tanh(x):
    x2 = x * x
    ser = x * (1.0 + x2 * (-0.3333333333 + x2 * (0.13333334 + x2 * -0.05396825)))
    t = jnp.exp(2.0 * x)
    big = (t - 1.0) / (t + 1.0)
    return jnp.where(jnp.abs(x) < 0.1, ser, big)


def _proj_scale(n):
    return jnp.where(n >= 1.0, 1.0 / (n - 1e-5), jnp.float32(1.0))


_EBLK = 4096


def _pack_body(x_ref, o_ref):
    eye = jax.lax.broadcasted_iota(jnp.int32, (DIM, DIM), 0) == \
        jax.lax.broadcasted_iota(jnp.int32, (DIM, DIM), 1)
    y = jax.lax.dot_general(x_ref[...], eye.astype(jnp.float32),
                            (((0,), (0,)), ((), ())),
                            preferred_element_type=jnp.float32)
    y3 = y.reshape(_EBLK // PACK, PACK, DIM)
    o_ref[...] = jnp.concatenate([y3[:, p, :] for p in range(PACK)], axis=1)


def _pack_entities(eh_t):
    grid = pl.cdiv(NUM_ENT, _EBLK)
    return pl.pallas_call(
        _pack_body,
        grid=(grid,),
        in_specs=[pl.BlockSpec((DIM, _EBLK), lambda g: (0, g))],
        out_specs=pl.BlockSpec((_EBLK // PACK, PACK * DIM),
                               lambda g: (g, 0)),
        out_shape=jax.ShapeDtypeStruct((NUM_ENT // PACK, PACK * DIM),
                                       jnp.float32),
    )(eh_t)


def _murp_body(u_idx_h, r_idx_h, v_idx_h, eh2_h, rvh_h, wu_h, bs_h, bo_h,
               out_h, uidx_v, ridx_v, vidx_v, ush_v, vsh_v, rch_v,
               u_rows, v_rows, w_rows, r_rows, bsu_v, bov_v, out_v, sem):
    wid = lax.axis_index("s") * _NC + lax.axis_index("c")
    base = wid * _BPW

    pltpu.sync_copy(u_idx_h.at[pl.ds(base, _BPW)], uidx_v)
    pltpu.sync_copy(r_idx_h.at[pl.ds(base, _BPW)], ridx_v)
    pltpu.sync_copy(v_idx_h.at[pl.ds(base, _BPW)], vidx_v)

    bcp = [pltpu.async_copy(bs_h.at[uidx_v], bsu_v, sem),
           pltpu.async_copy(bo_h.at[vidx_v], bov_v, sem)]

    for k in range(_BPW // _BLK):
        ch, off = k // _BPC, (k % _BPC) * _BLK
        sl = pl.ds(k * _BLK, _BLK)
        ush_v[ch, pl.ds(off, _BLK)] = lax.shift_right_logical(
            uidx_v[sl], jnp.int32(2))
        vsh_v[ch, pl.ds(off, _BLK)] = lax.shift_right_logical(
            vidx_v[sl], jnp.int32(2))
        rch_v[ch, pl.ds(off, _BLK)] = ridx_v[sl]

    lane = lax.broadcasted_iota(jnp.int32, (_BLK,), 0)

    for ch in range(_NCH):
        cps = [
            pltpu.async_copy(eh2_h.at[ush_v.at[ch]], u_rows, sem),
            pltpu.async_copy(eh2_h.at[vsh_v.at[ch]], v_rows, sem),
            pltpu.async_copy(wu_h.at[rch_v.at[ch]], w_rows, sem),
            pltpu.async_copy(rvh_h.at[rch_v.at[ch]], r_rows, sem),
        ]
        for cp in cps:
            cp.wait()
        if ch == 0:
            for cp in bcp:
                cp.wait()

        def block(blk, carry, ch=ch):
            b16 = blk * _BLK
            rows = b16 + lane
            uidx16 = uidx_v[pl.ds(ch * _CH + b16, _BLK)]
            vidx16 = vidx_v[pl.ds(ch * _CH + b16, _BLK)]
            ucol0 = lax.shift_left(lax.bitwise_and(uidx16, jnp.int32(3)),
                                   jnp.int32(5))
            vcol0 = lax.shift_left(lax.bitwise_and(vidx16, jnp.int32(3)),
                                   jnp.int32(5))
            zero = jnp.zeros((_BLK,), jnp.float32)
            A = zero; Bv = zero; C = zero; D = zero; E = zero; F = zero
            G = zero
            for j in range(DIM):
                col = jnp.full((_BLK,), j, jnp.int32)
                uj = plsc.load_gather(u_rows, [rows, ucol0 + j])
                vj = plsc.load_gather(v_rows, [rows, vcol0 + j])
                wj = plsc.load_gather(w_rows, [rows, col])
                rj = plsc.load_gather(r_rows, [rows, col])
                mj = uj * wj
                A = A + uj * uj
                Bv = Bv + vj * vj
                C = C + rj * rj
                D = D + mj * mj
                E = E + mj * vj
                F = F + mj * rj
                G = G + vj * rj

            nu = _soft_sqrt(A)
            nv = _soft_sqrt(Bv)
            nr = _soft_sqrt(C)
            su = _proj_scale(nu)
            sv = _proj_scale(nv)
            sr = _proj_scale(nr)
            n1 = jnp.clip(su * nu, 1e-10, 1.0 - 1e-5)
            alpha = _soft_artanh(n1) / n1 * su
            nm = _soft_sqrt(D)
            n2 = jnp.maximum(alpha * nm, 1e-10)
            beta = _soft_tanh(n2) / n2 * alpha
            sqx = jnp.clip(sv * sv * Bv, 0.0, 1.0 - 1e-5)
            sqy = jnp.clip(sr * sr * C, 0.0, 1.0 - 1e-5)
            dxy = sv * sr * G
            den = 1.0 + 2.0 * dxy + sqx * sqy
            a = sv * (1.0 + 2.0 * dxy + sqy) / den
            b = sr * (1.0 - sqx) / den
            p = _proj_scale(beta * nm) * beta
            Q = a * a * Bv + 2.0 * a * b * G + b * b * C
            s4 = _proj_scale(_soft_sqrt(Q))
            a2 = s4 * a
            b2 = s4 * b
            sqx2 = jnp.clip(p * p * D, 0.0, 1.0 - 1e-5)
            sqy2 = jnp.clip(s4 * s4 * Q, 0.0, 1.0 - 1e-5)
            dxy2 = -p * (a2 * E + b2 * F)
            den2 = 1.0 + 2.0 * dxy2 + sqx2 * sqy2
            cm = -(1.0 + 2.0 * dxy2 + sqy2) * p / den2
            cv = (1.0 - sqx2) * a2 / den2
            cr = (1.0 - sqx2) * b2 / den2
            z2 = (cm * cm * D + cv * cv * Bv + cr * cr * C
                  + 2.0 * cm * cv * E + 2.0 * cm * cr * F
                  + 2.0 * cv * cr * G)
            nz = jnp.clip(_soft_sqrt(z2), 1e-10, 1.0 - 1e-5)
            at = _soft_artanh(nz)
            sq = 4.0 * at * at
            res = (-sq + bsu_v[pl.ds(ch * _CH + b16, _BLK)]
                   + bov_v[pl.ds(ch * _CH + b16, _BLK)])
            out_v[pl.ds(ch * _CH + b16, _BLK)] = res
            return carry

        lax.fori_loop(0, _BPC, block, 0)

    pltpu.sync_copy(out_v, out_h.at[pl.ds(base, _BPW)])


@jax.jit
def kernel(u_idx, r_idx, v_idx, Eh, rvh, Wu, bs, bo):
    mesh = plsc.VectorSubcoreMesh(core_axis_name="c", subcore_axis_name="s")
    run = pl.kernel(
        _murp_body,
        out_type=jax.ShapeDtypeStruct((B,), jnp.float32),
        mesh=mesh,
        scratch_types=[
            pltpu.VMEM((_BPW,), jnp.int32),
            pltpu.VMEM((_BPW,), jnp.int32),
            pltpu.VMEM((_BPW,), jnp.int32),
            pltpu.VMEM((_NCH, _CH), jnp.int32),
            pltpu.VMEM((_NCH, _CH), jnp.int32),
            pltpu.VMEM((_NCH, _CH), jnp.int32),
            pltpu.VMEM((_CH, PACK * DIM), jnp.float32),
            pltpu.VMEM((_CH, PACK * DIM), jnp.float32),
            pltpu.VMEM((_CH, DIM), jnp.float32),
            pltpu.VMEM((_CH, DIM), jnp.float32),
            pltpu.VMEM((_BPW,), jnp.float32),
            pltpu.VMEM((_BPW,), jnp.float32),
            pltpu.VMEM((_BPW,), jnp.float32),
            pltpu.SemaphoreType.DMA,
        ],
        compiler_params=pltpu.CompilerParams(
            needs_layout_passes=False, use_tc_tiling_on_sc=False),
    )
    eh2 = _pack_entities(jnp.swapaxes(Eh, 0, 1))
    return run(u_idx.astype(jnp.int32), r_idx.astype(jnp.int32),
               v_idx.astype(jnp.int32), eh2, rvh, Wu, bs, bo)


# speedup vs baseline: 1.3855x; 1.3855x over previous
import functools

import jax
import jax.numpy as jnp
from jax import lax
from jax.experimental import pallas as pl
from jax.experimental.pallas import tpu as pltpu
from jax.experimental.pallas import tpu_sc as plsc

NUM_ENT = 1000000
NUM_REL = 1000
DIM = 32
B = 16384
PACK = 4

_NC, _NS = 2, 16
_NW = _NC * _NS
_BPW = B // _NW
_BLK = 16
_CH = 128
_NCH = _BPW // _CH
_BPC = _CH // _BLK


def _soft_rsqrt(x):
    i = lax.bitcast_convert_type(x, jnp.int32)
    i = jnp.int32(0x5F3759DF) - lax.shift_right_arithmetic(i, jnp.int32(1))
    y = lax.bitcast_convert_type(i, jnp.float32)
    for _ in range(3):
        y = y * (1.5 - 0.5 * x * y * y)
    return y


def _soft_sqrt(x):
    return x * _soft_rsqrt(x)


def _soft_log(x):
    i = lax.bitcast_convert_type(x, jnp.int32)
    e = lax.shift_right_arithmetic(i, jnp.int32(23)) - jnp.int32(127)
    mbits = lax.bitwise_or(lax.bitwise_and(i, jnp.int32(0x007FFFFF)),
                           jnp.int32(0x3F800000))
    m = lax.bitcast_convert_type(mbits, jnp.float32)
    big = m > 1.4142135381698608
    m = jnp.where(big, m * 0.5, m)
    e = jnp.where(big, e + jnp.int32(1), e)
    ef = e.astype(jnp.float32)
    t = (m - 1.0) / (m + 1.0)
    t2 = t * t
    p = 1.0 + t2 * (0.3333333333 + t2 * (0.2 + t2 * (0.14285714 + t2 * 0.11111111)))
    return ef * 0.6931471805599453 + 2.0 * t * p


def _soft_artanh(x):
    return 0.5 * _soft_log((1.0 + x) / (1.0 - x))


def _soft_tanh(x):
    x2 = x * x
    ser = x * (1.0 + x2 * (-0.3333333333 + x2 * (0.13333334 + x2 * -0.05396825)))
    t = jnp.exp(2.0 * x)
    big = (t - 1.0) / (t + 1.0)
    return jnp.where(jnp.abs(x) < 0.1, ser, big)


def _proj_scale(n):
    return jnp.where(n >= 1.0, 1.0 / (n - 1e-5), jnp.float32(1.0))


_EBLK = 4096
_QCH = _EBLK // PACK
_NROWS = ((NUM_ENT + _EBLK - 1) // _EBLK) * _QCH


def _pack_body(x_ref, o_ref):
    y = x_ref[...].T
    q = _EBLK // PACK
    o_ref[...] = jnp.concatenate(
        [y[p * q:(p + 1) * q, :] for p in range(PACK)], axis=1)


def _pack_entities(eh_t):
    grid = pl.cdiv(NUM_ENT, _EBLK)
    return pl.pallas_call(
        _pack_body,
        grid=(grid,),
        in_specs=[pl.BlockSpec((DIM, _EBLK), lambda g: (0, g))],
        out_specs=pl.BlockSpec((_QCH, PACK * DIM), lambda g: (g, 0)),
        out_shape=jax.ShapeDtypeStruct((_NROWS, PACK * DIM), jnp.float32),
    )(eh_t)


def _murp_body(u_idx_h, r_idx_h, v_idx_h, eh2_h, rvh_h, wu_h, bs_h, bo_h,
               out_h, uidx_v, ridx_v, vidx_v, ush_v, vsh_v, rch_v,
               u_rows, v_rows, w_rows, r_rows, bsu_v, bov_v, out_v, sem):
    wid = lax.axis_index("s") * _NC + lax.axis_index("c")
    base = wid * _BPW

    pltpu.sync_copy(u_idx_h.at[pl.ds(base, _BPW)], uidx_v)
    pltpu.sync_copy(r_idx_h.at[pl.ds(base, _BPW)], ridx_v)
    pltpu.sync_copy(v_idx_h.at[pl.ds(base, _BPW)], vidx_v)

    bcp = [pltpu.async_copy(bs_h.at[uidx_v], bsu_v, sem),
           pltpu.async_copy(bo_h.at[vidx_v], bov_v, sem)]

    for k in range(_BPW // _BLK):
        ch, off = k // _BPC, (k % _BPC) * _BLK
        sl = pl.ds(k * _BLK, _BLK)
        ui = uidx_v[sl]
        vi = vidx_v[sl]
        ush_v[ch, pl.ds(off, _BLK)] = lax.bitwise_or(
            lax.shift_left(lax.shift_right_logical(ui, jnp.int32(12)),
                           jnp.int32(10)),
            lax.bitwise_and(ui, jnp.int32(1023)))
        vsh_v[ch, pl.ds(off, _BLK)] = lax.bitwise_or(
            lax.shift_left(lax.shift_right_logical(vi, jnp.int32(12)),
                           jnp.int32(10)),
            lax.bitwise_and(vi, jnp.int32(1023)))
        rch_v[ch, pl.ds(off, _BLK)] = ridx_v[sl]

    lane = lax.broadcasted_iota(jnp.int32, (_BLK,), 0)

    for ch in range(_NCH):
        cps = [
            pltpu.async_copy(eh2_h.at[ush_v.at[ch]], u_rows, sem),
            pltpu.async_copy(eh2_h.at[vsh_v.at[ch]], v_rows, sem),
            pltpu.async_copy(wu_h.at[rch_v.at[ch]], w_rows, sem),
            pltpu.async_copy(rvh_h.at[rch_v.at[ch]], r_rows, sem),
        ]
        for cp in cps:
            cp.wait()
        if ch == 0:
            for cp in bcp:
                cp.wait()

        def block(blk, carry, ch=ch):
            b16 = blk * _BLK
            rows = b16 + lane
            uidx16 = uidx_v[pl.ds(ch * _CH + b16, _BLK)]
            vidx16 = vidx_v[pl.ds(ch * _CH + b16, _BLK)]
            ucol0 = lax.shift_left(
                lax.bitwise_and(lax.shift_right_logical(uidx16, jnp.int32(10)),
                                jnp.int32(3)), jnp.int32(5))
            vcol0 = lax.shift_left(
                lax.bitwise_and(lax.shift_right_logical(vidx16, jnp.int32(10)),
                                jnp.int32(3)), jnp.int32(5))
            zero = jnp.zeros((_BLK,), jnp.float32)
            A = zero; Bv = zero; C = zero; D = zero; E = zero; F = zero
            G = zero
            for j in range(DIM):
                col = jnp.full((_BLK,), j, jnp.int32)
                uj = plsc.load_gather(u_rows, [rows, ucol0 + j])
                vj = plsc.load_gather(v_rows, [rows, vcol0 + j])
                wj = plsc.load_gather(w_rows, [rows, col])
                rj = plsc.load_gather(r_rows, [rows, col])
                mj = uj * wj
                A = A + uj * uj
                Bv = Bv + vj * vj
                C = C + rj * rj
                D = D + mj * mj
                E = E + mj * vj
                F = F + mj * rj
                G = G + vj * rj

            nu = _soft_sqrt(A)
            nv = _soft_sqrt(Bv)
            nr = _soft_sqrt(C)
            su = _proj_scale(nu)
            sv = _proj_scale(nv)
            sr = _proj_scale(nr)
            n1 = jnp.clip(su * nu, 1e-10, 1.0 - 1e-5)
            alpha = _soft_artanh(n1) / n1 * su
            nm = _soft_sqrt(D)
            n2 = jnp.maximum(alpha * nm, 1e-10)
            beta = _soft_tanh(n2) / n2 * alpha
            sqx = jnp.clip(sv * sv * Bv, 0.0, 1.0 - 1e-5)
            sqy = jnp.clip(sr * sr * C, 0.0, 1.0 - 1e-5)
            dxy = sv * sr * G
            den = 1.0 + 2.0 * dxy + sqx * sqy
            a = sv * (1.0 + 2.0 * dxy + sqy) / den
            b = sr * (1.0 - sqx) / den
            p = _proj_scale(beta * nm) * beta
            Q = a * a * Bv + 2.0 * a * b * G + b * b * C
            s4 = _proj_scale(_soft_sqrt(Q))
            a2 = s4 * a
            b2 = s4 * b
            sqx2 = jnp.clip(p * p * D, 0.0, 1.0 - 1e-5)
            sqy2 = jnp.clip(s4 * s4 * Q, 0.0, 1.0 - 1e-5)
            dxy2 = -p * (a2 * E + b2 * F)
            den2 = 1.0 + 2.0 * dxy2 + sqx2 * sqy2
            cm = -(1.0 + 2.0 * dxy2 + sqy2) * p / den2
            cv = (1.0 - sqx2) * a2 / den2
            cr = (1.0 - sqx2) * b2 / den2
            z2 = (cm * cm * D + cv * cv * Bv + cr * cr * C
                  + 2.0 * cm * cv * E + 2.0 * cm * cr * F
                  + 2.0 * cv * cr * G)
            nz = jnp.clip(_soft_sqrt(z2), 1e-10, 1.0 - 1e-5)
            at = _soft_artanh(nz)
            sq = 4.0 * at * at
            res = (-sq + bsu_v[pl.ds(ch * _CH + b16, _BLK)]
                   + bov_v[pl.ds(ch * _CH + b16, _BLK)])
            out_v[pl.ds(ch * _CH + b16, _BLK)] = res
            return carry

        lax.fori_loop(0, _BPC, block, 0)

    pltpu.sync_copy(out_v, out_h.at[pl.ds(base, _BPW)])


@jax.jit
def kernel(u_idx, r_idx, v_idx, Eh, rvh, Wu, bs, bo):
    mesh = plsc.VectorSubcoreMesh(core_axis_name="c", subcore_axis_name="s")
    run = pl.kernel(
        _murp_body,
        out_type=jax.ShapeDtypeStruct((B,), jnp.float32),
        mesh=mesh,
        scratch_types=[
            pltpu.VMEM((_BPW,), jnp.int32),
            pltpu.VMEM((_BPW,), jnp.int32),
            pltpu.VMEM((_BPW,), jnp.int32),
            pltpu.VMEM((_NCH, _CH), jnp.int32),
            pltpu.VMEM((_NCH, _CH), jnp.int32),
            pltpu.VMEM((_NCH, _CH), jnp.int32),
            pltpu.VMEM((_CH, PACK * DIM), jnp.float32),
            pltpu.VMEM((_CH, PACK * DIM), jnp.float32),
            pltpu.VMEM((_CH, DIM), jnp.float32),
            pltpu.VMEM((_CH, DIM), jnp.float32),
            pltpu.VMEM((_BPW,), jnp.float32),
            pltpu.VMEM((_BPW,), jnp.float32),
            pltpu.VMEM((_BPW,), jnp.float32),
            pltpu.SemaphoreType.DMA,
        ],
        compiler_params=pltpu.CompilerParams(
            needs_layout_passes=False, use_tc_tiling_on_sc=False),
    )
    eh2 = _pack_entities(jnp.swapaxes(Eh, 0, 1))
    return run(u_idx.astype(jnp.int32), r_idx.astype(jnp.int32),
               v_idx.astype(jnp.int32), eh2, rvh, Wu, bs, bo)


# speedup vs baseline: 1.5799x; 1.1403x over previous
import functools

import jax
import jax.numpy as jnp
from jax import lax
from jax.experimental import pallas as pl
from jax.experimental.pallas import tpu as pltpu
from jax.experimental.pallas import tpu_sc as plsc

NUM_ENT = 1000000
NUM_REL = 1000
DIM = 32
B = 16384
PACK = 4

_NC, _NS = 2, 16
_NW = _NC * _NS
_BPW = B // _NW
_BLK = 16
_CH = 128
_NCH = _BPW // _CH
_BPC = _CH // _BLK


def _soft_rsqrt(x):
    i = lax.bitcast_convert_type(x, jnp.int32)
    i = jnp.int32(0x5F3759DF) - lax.shift_right_arithmetic(i, jnp.int32(1))
    y = lax.bitcast_convert_type(i, jnp.float32)
    for _ in range(3):
        y = y * (1.5 - 0.5 * x * y * y)
    return y


def _soft_sqrt(x):
    return x * _soft_rsqrt(x)


def _soft_log(x):
    i = lax.bitcast_convert_type(x, jnp.int32)
    e = lax.shift_right_arithmetic(i, jnp.int32(23)) - jnp.int32(127)
    mbits = lax.bitwise_or(lax.bitwise_and(i, jnp.int32(0x007FFFFF)),
                           jnp.int32(0x3F800000))
    m = lax.bitcast_convert_type(mbits, jnp.float32)
    big = m > 1.4142135381698608
    m = jnp.where(big, m * 0.5, m)
    e = jnp.where(big, e + jnp.int32(1), e)
    ef = e.astype(jnp.float32)
    t = (m - 1.0) / (m + 1.0)
    t2 = t * t
    p = 1.0 + t2 * (0.3333333333 + t2 * (0.2 + t2 * (0.14285714 + t2 * 0.11111111)))
    return ef * 0.6931471805599453 + 2.0 * t * p


def _soft_artanh(x):
    return 0.5 * _soft_log((1.0 + x) / (1.0 - x))


def _soft_tanh(x):
    x2 = x * x
    ser = x * (1.0 + x2 * (-0.3333333333 + x2 * (0.13333334 + x2 * -0.05396825)))
    t = jnp.exp(2.0 * x)
    big = (t - 1.0) / (t + 1.0)
    return jnp.where(jnp.abs(x) < 0.1, ser, big)


def _proj_scale(n):
    return jnp.where(n >= 1.0, 1.0 / (n - 1e-5), jnp.float32(1.0))


_EBLK = 8192
_SH_G = 13
_SH_Q = 11
_QCH = _EBLK // PACK
_NROWS = ((NUM_ENT + _EBLK - 1) // _EBLK) * _QCH


def _pack_body(x_ref, o_ref):
    y = x_ref[...].T
    q = _EBLK // PACK
    o_ref[...] = jnp.concatenate(
        [y[p * q:(p + 1) * q, :] for p in range(PACK)], axis=1)


def _pack_entities(eh_t):
    grid = pl.cdiv(NUM_ENT, _EBLK)
    return pl.pallas_call(
        _pack_body,
        grid=(grid,),
        in_specs=[pl.BlockSpec((DIM, _EBLK), lambda g: (0, g))],
        out_specs=pl.BlockSpec((_QCH, PACK * DIM), lambda g: (g, 0)),
        out_shape=jax.ShapeDtypeStruct((_NROWS, PACK * DIM), jnp.float32),
    )(eh_t)


def _murp_body(u_idx_h, r_idx_h, v_idx_h, eh2_h, rvh_h, wu_h, bs_h, bo_h,
               out_h, uidx_v, ridx_v, vidx_v, ush_v, vsh_v, rch_v,
               u_rows, v_rows, w_rows, r_rows, bsu_v, bov_v, out_v, sem):
    wid = lax.axis_index("s") * _NC + lax.axis_index("c")
    base = wid * _BPW

    pltpu.sync_copy(u_idx_h.at[pl.ds(base, _BPW)], uidx_v)
    pltpu.sync_copy(r_idx_h.at[pl.ds(base, _BPW)], ridx_v)
    pltpu.sync_copy(v_idx_h.at[pl.ds(base, _BPW)], vidx_v)

    bcp = [pltpu.async_copy(bs_h.at[uidx_v], bsu_v, sem),
           pltpu.async_copy(bo_h.at[vidx_v], bov_v, sem)]

    for k in range(_BPW // _BLK):
        ch, off = k // _BPC, (k % _BPC) * _BLK
        sl = pl.ds(k * _BLK, _BLK)
        ui = uidx_v[sl]
        vi = vidx_v[sl]
        ush_v[ch, pl.ds(off, _BLK)] = lax.bitwise_or(
            lax.shift_left(lax.shift_right_logical(ui, jnp.int32(_SH_G)),
                           jnp.int32(_SH_Q)),
            lax.bitwise_and(ui, jnp.int32(_QCH - 1)))
        vsh_v[ch, pl.ds(off, _BLK)] = lax.bitwise_or(
            lax.shift_left(lax.shift_right_logical(vi, jnp.int32(_SH_G)),
                           jnp.int32(_SH_Q)),
            lax.bitwise_and(vi, jnp.int32(_QCH - 1)))
        rch_v[ch, pl.ds(off, _BLK)] = ridx_v[sl]

    lane = lax.broadcasted_iota(jnp.int32, (_BLK,), 0)

    for ch in range(_NCH):
        cps = [
            pltpu.async_copy(eh2_h.at[ush_v.at[ch]], u_rows, sem),
            pltpu.async_copy(eh2_h.at[vsh_v.at[ch]], v_rows, sem),
            pltpu.async_copy(wu_h.at[rch_v.at[ch]], w_rows, sem),
            pltpu.async_copy(rvh_h.at[rch_v.at[ch]], r_rows, sem),
        ]
        for cp in cps:
            cp.wait()
        if ch == 0:
            for cp in bcp:
                cp.wait()

        def block(blk, carry, ch=ch):
            b16 = blk * _BLK
            rows = b16 + lane
            uidx16 = uidx_v[pl.ds(ch * _CH + b16, _BLK)]
            vidx16 = vidx_v[pl.ds(ch * _CH + b16, _BLK)]
            ucol0 = lax.shift_left(
                lax.bitwise_and(lax.shift_right_logical(uidx16, jnp.int32(_SH_Q)),
                                jnp.int32(3)), jnp.int32(5))
            vcol0 = lax.shift_left(
                lax.bitwise_and(lax.shift_right_logical(vidx16, jnp.int32(_SH_Q)),
                                jnp.int32(3)), jnp.int32(5))
            zero = jnp.zeros((_BLK,), jnp.float32)
            A = zero; Bv = zero; C = zero; D = zero; E = zero; F = zero
            G = zero
            for j in range(DIM):
                col = jnp.full((_BLK,), j, jnp.int32)
                uj = plsc.load_gather(u_rows, [rows, ucol0 + j])
                vj = plsc.load_gather(v_rows, [rows, vcol0 + j])
                wj = plsc.load_gather(w_rows, [rows, col])
                rj = plsc.load_gather(r_rows, [rows, col])
                mj = uj * wj
                A = A + uj * uj
                Bv = Bv + vj * vj
                C = C + rj * rj
                D = D + mj * mj
                E = E + mj * vj
                F = F + mj * rj
                G = G + vj * rj

            nu = _soft_sqrt(A)
            nv = _soft_sqrt(Bv)
            nr = _soft_sqrt(C)
            su = _proj_scale(nu)
            sv = _proj_scale(nv)
            sr = _proj_scale(nr)
            n1 = jnp.clip(su * nu, 1e-10, 1.0 - 1e-5)
            alpha = _soft_artanh(n1) / n1 * su
            nm = _soft_sqrt(D)
            n2 = jnp.maximum(alpha * nm, 1e-10)
            beta = _soft_tanh(n2) / n2 * alpha
            sqx = jnp.clip(sv * sv * Bv, 0.0, 1.0 - 1e-5)
            sqy = jnp.clip(sr * sr * C, 0.0, 1.0 - 1e-5)
            dxy = sv * sr * G
            den = 1.0 + 2.0 * dxy + sqx * sqy
            a = sv * (1.0 + 2.0 * dxy + sqy) / den
            b = sr * (1.0 - sqx) / den
            p = _proj_scale(beta * nm) * beta
            Q = a * a * Bv + 2.0 * a * b * G + b * b * C
            s4 = _proj_scale(_soft_sqrt(Q))
            a2 = s4 * a
            b2 = s4 * b
            sqx2 = jnp.clip(p * p * D, 0.0, 1.0 - 1e-5)
            sqy2 = jnp.clip(s4 * s4 * Q, 0.0, 1.0 - 1e-5)
            dxy2 = -p * (a2 * E + b2 * F)
            den2 = 1.0 + 2.0 * dxy2 + sqx2 * sqy2
            cm = -(1.0 + 2.0 * dxy2 + sqy2) * p / den2
            cv = (1.0 - sqx2) * a2 / den2
            cr = (1.0 - sqx2) * b2 / den2
            z2 = (cm * cm * D + cv * cv * Bv + cr * cr * C
                  + 2.0 * cm * cv * E + 2.0 * cm * cr * F
                  + 2.0 * cv * cr * G)
            nz = jnp.clip(_soft_sqrt(z2), 1e-10, 1.0 - 1e-5)
            at = _soft_artanh(nz)
            sq = 4.0 * at * at
            res = (-sq + bsu_v[pl.ds(ch * _CH + b16, _BLK)]
                   + bov_v[pl.ds(ch * _CH + b16, _BLK)])
            out_v[pl.ds(ch * _CH + b16, _BLK)] = res
            return carry

        lax.fori_loop(0, _BPC, block, 0)

    pltpu.sync_copy(out_v, out_h.at[pl.ds(base, _BPW)])


@jax.jit
def kernel(u_idx, r_idx, v_idx, Eh, rvh, Wu, bs, bo):
    mesh = plsc.VectorSubcoreMesh(core_axis_name="c", subcore_axis_name="s")
    run = pl.kernel(
        _murp_body,
        out_type=jax.ShapeDtypeStruct((B,), jnp.float32),
        mesh=mesh,
        scratch_types=[
            pltpu.VMEM((_BPW,), jnp.int32),
            pltpu.VMEM((_BPW,), jnp.int32),
            pltpu.VMEM((_BPW,), jnp.int32),
            pltpu.VMEM((_NCH, _CH), jnp.int32),
            pltpu.VMEM((_NCH, _CH), jnp.int32),
            pltpu.VMEM((_NCH, _CH), jnp.int32),
            pltpu.VMEM((_CH, PACK * DIM), jnp.float32),
            pltpu.VMEM((_CH, PACK * DIM), jnp.float32),
            pltpu.VMEM((_CH, DIM), jnp.float32),
            pltpu.VMEM((_CH, DIM), jnp.float32),
            pltpu.VMEM((_BPW,), jnp.float32),
            pltpu.VMEM((_BPW,), jnp.float32),
            pltpu.VMEM((_BPW,), jnp.float32),
            pltpu.SemaphoreType.DMA,
        ],
        compiler_params=pltpu.CompilerParams(
            needs_layout_passes=False, use_tc_tiling_on_sc=False),
    )
    eh2 = _pack_entities(jnp.swapaxes(Eh, 0, 1))
    return run(u_idx.astype(jnp.int32), r_idx.astype(jnp.int32),
               v_idx.astype(jnp.int32), eh2, rvh, Wu, bs, bo)


# speedup vs baseline: 1.5998x; 1.0126x over previous
import functools

import jax
import jax.numpy as jnp
from jax import lax
from jax.experimental import pallas as pl
from jax.experimental.pallas import tpu as pltpu
from jax.experimental.pallas import tpu_sc as plsc

NUM_ENT = 1000000
NUM_REL = 1000
DIM = 32
B = 16384
PACK = 4

_NC, _NS = 2, 16
_NW = _NC * _NS
_BPW = B // _NW
_BLK = 16
_CH = 128
_NCH = _BPW // _CH
_BPC = _CH // _BLK


def _soft_rsqrt(x):
    i = lax.bitcast_convert_type(x, jnp.int32)
    i = jnp.int32(0x5F3759DF) - lax.shift_right_arithmetic(i, jnp.int32(1))
    y = lax.bitcast_convert_type(i, jnp.float32)
    for _ in range(3):
        y = y * (1.5 - 0.5 * x * y * y)
    return y


def _soft_sqrt(x):
    return x * _soft_rsqrt(x)


def _soft_log(x):
    i = lax.bitcast_convert_type(x, jnp.int32)
    e = lax.shift_right_arithmetic(i, jnp.int32(23)) - jnp.int32(127)
    mbits = lax.bitwise_or(lax.bitwise_and(i, jnp.int32(0x007FFFFF)),
                           jnp.int32(0x3F800000))
    m = lax.bitcast_convert_type(mbits, jnp.float32)
    big = m > 1.4142135381698608
    m = jnp.where(big, m * 0.5, m)
    e = jnp.where(big, e + jnp.int32(1), e)
    ef = e.astype(jnp.float32)
    t = (m - 1.0) / (m + 1.0)
    t2 = t * t
    p = 1.0 + t2 * (0.3333333333 + t2 * (0.2 + t2 * (0.14285714 + t2 * 0.11111111)))
    return ef * 0.6931471805599453 + 2.0 * t * p


def _soft_artanh(x):
    return 0.5 * _soft_log((1.0 + x) / (1.0 - x))


def _soft_tanh(x):
    x2 = x * x
    ser = x * (1.0 + x2 * (-0.3333333333 + x2 * (0.13333334 + x2 * -0.05396825)))
    t = jnp.exp(2.0 * x)
    big = (t - 1.0) / (t + 1.0)
    return jnp.where(jnp.abs(x) < 0.1, ser, big)


def _proj_scale(n):
    return jnp.where(n >= 1.0, 1.0 / (n - 1e-5), jnp.float32(1.0))


_EBLK = 16384
_SH_G = 14
_SH_Q = 12
_QCH = _EBLK // PACK
_NROWS = ((NUM_ENT + _EBLK - 1) // _EBLK) * _QCH


def _pack_body(x_ref, o_ref):
    y = x_ref[...].T
    q = _EBLK // PACK
    o_ref[...] = jnp.concatenate(
        [y[p * q:(p + 1) * q, :] for p in range(PACK)], axis=1)


def _pack_entities(eh_t):
    grid = pl.cdiv(NUM_ENT, _EBLK)
    return pl.pallas_call(
        _pack_body,
        grid=(grid,),
        in_specs=[pl.BlockSpec((DIM, _EBLK), lambda g: (0, g))],
        out_specs=pl.BlockSpec((_QCH, PACK * DIM), lambda g: (g, 0)),
        out_shape=jax.ShapeDtypeStruct((_NROWS, PACK * DIM), jnp.float32),
    )(eh_t)


def _murp_body(u_idx_h, r_idx_h, v_idx_h, eh2_h, rvh_h, wu_h, bs_h, bo_h,
               out_h, uidx_v, ridx_v, vidx_v, ush_v, vsh_v, rch_v,
               u_rows, v_rows, w_rows, r_rows, bsu_v, bov_v, out_v, sem):
    wid = lax.axis_index("s") * _NC + lax.axis_index("c")
    base = wid * _BPW

    pltpu.sync_copy(u_idx_h.at[pl.ds(base, _BPW)], uidx_v)
    pltpu.sync_copy(r_idx_h.at[pl.ds(base, _BPW)], ridx_v)
    pltpu.sync_copy(v_idx_h.at[pl.ds(base, _BPW)], vidx_v)

    bcp = [pltpu.async_copy(bs_h.at[uidx_v], bsu_v, sem),
           pltpu.async_copy(bo_h.at[vidx_v], bov_v, sem)]

    for k in range(_BPW // _BLK):
        ch, off = k // _BPC, (k % _BPC) * _BLK
        sl = pl.ds(k * _BLK, _BLK)
        ui = uidx_v[sl]
        vi = vidx_v[sl]
        ush_v[ch, pl.ds(off, _BLK)] = lax.bitwise_or(
            lax.shift_left(lax.shift_right_logical(ui, jnp.int32(_SH_G)),
                           jnp.int32(_SH_Q)),
            lax.bitwise_and(ui, jnp.int32(_QCH - 1)))
        vsh_v[ch, pl.ds(off, _BLK)] = lax.bitwise_or(
            lax.shift_left(lax.shift_right_logical(vi, jnp.int32(_SH_G)),
                           jnp.int32(_SH_Q)),
            lax.bitwise_and(vi, jnp.int32(_QCH - 1)))
        rch_v[ch, pl.ds(off, _BLK)] = ridx_v[sl]

    lane = lax.broadcasted_iota(jnp.int32, (_BLK,), 0)

    for ch in range(_NCH):
        cps = [
            pltpu.async_copy(eh2_h.at[ush_v.at[ch]], u_rows, sem),
            pltpu.async_copy(eh2_h.at[vsh_v.at[ch]], v_rows, sem),
            pltpu.async_copy(wu_h.at[rch_v.at[ch]], w_rows, sem),
            pltpu.async_copy(rvh_h.at[rch_v.at[ch]], r_rows, sem),
        ]
        for cp in cps:
            cp.wait()
        if ch == 0:
            for cp in bcp:
                cp.wait()

        def block(blk, carry, ch=ch):
            b16 = blk * _BLK
            rows = b16 + lane
            uidx16 = uidx_v[pl.ds(ch * _CH + b16, _BLK)]
            vidx16 = vidx_v[pl.ds(ch * _CH + b16, _BLK)]
            ucol0 = lax.shift_left(
                lax.bitwise_and(lax.shift_right_logical(uidx16, jnp.int32(_SH_Q)),
                                jnp.int32(3)), jnp.int32(5))
            vcol0 = lax.shift_left(
                lax.bitwise_and(lax.shift_right_logical(vidx16, jnp.int32(_SH_Q)),
                                jnp.int32(3)), jnp.int32(5))
            zero = jnp.zeros((_BLK,), jnp.float32)
            A = zero; Bv = zero; C = zero; D = zero; E = zero; F = zero
            G = zero
            for j in range(DIM):
                col = jnp.full((_BLK,), j, jnp.int32)
                uj = plsc.load_gather(u_rows, [rows, ucol0 + j])
                vj = plsc.load_gather(v_rows, [rows, vcol0 + j])
                wj = plsc.load_gather(w_rows, [rows, col])
                rj = plsc.load_gather(r_rows, [rows, col])
                mj = uj * wj
                A = A + uj * uj
                Bv = Bv + vj * vj
                C = C + rj * rj
                D = D + mj * mj
                E = E + mj * vj
                F = F + mj * rj
                G = G + vj * rj

            nu = _soft_sqrt(A)
            nv = _soft_sqrt(Bv)
            nr = _soft_sqrt(C)
            su = _proj_scale(nu)
            sv = _proj_scale(nv)
            sr = _proj_scale(nr)
            n1 = jnp.clip(su * nu, 1e-10, 1.0 - 1e-5)
            alpha = _soft_artanh(n1) / n1 * su
            nm = _soft_sqrt(D)
            n2 = jnp.maximum(alpha * nm, 1e-10)
            beta = _soft_tanh(n2) / n2 * alpha
            sqx = jnp.clip(sv * sv * Bv, 0.0, 1.0 - 1e-5)
            sqy = jnp.clip(sr * sr * C, 0.0, 1.0 - 1e-5)
            dxy = sv * sr * G
            den = 1.0 + 2.0 * dxy + sqx * sqy
            a = sv * (1.0 + 2.0 * dxy + sqy) / den
            b = sr * (1.0 - sqx) / den
            p = _proj_scale(beta * nm) * beta
            Q = a * a * Bv + 2.0 * a * b * G + b * b * C
            s4 = _proj_scale(_soft_sqrt(Q))
            a2 = s4 * a
            b2 = s4 * b
            sqx2 = jnp.clip(p * p * D, 0.0, 1.0 - 1e-5)
            sqy2 = jnp.clip(s4 * s4 * Q, 0.0, 1.0 - 1e-5)
            dxy2 = -p * (a2 * E + b2 * F)
            den2 = 1.0 + 2.0 * dxy2 + sqx2 * sqy2
            cm = -(1.0 + 2.0 * dxy2 + sqy2) * p / den2
            cv = (1.0 - sqx2) * a2 / den2
            cr = (1.0 - sqx2) * b2 / den2
            z2 = (cm * cm * D + cv * cv * Bv + cr * cr * C
                  + 2.0 * cm * cv * E + 2.0 * cm * cr * F
                  + 2.0 * cv * cr * G)
            nz = jnp.clip(_soft_sqrt(z2), 1e-10, 1.0 - 1e-5)
            at = _soft_artanh(nz)
            sq = 4.0 * at * at
            res = (-sq + bsu_v[pl.ds(ch * _CH + b16, _BLK)]
                   + bov_v[pl.ds(ch * _CH + b16, _BLK)])
            out_v[pl.ds(ch * _CH + b16, _BLK)] = res
            return carry

        lax.fori_loop(0, _BPC, block, 0)

    pltpu.sync_copy(out_v, out_h.at[pl.ds(base, _BPW)])


@jax.jit
def kernel(u_idx, r_idx, v_idx, Eh, rvh, Wu, bs, bo):
    mesh = plsc.VectorSubcoreMesh(core_axis_name="c", subcore_axis_name="s")
    run = pl.kernel(
        _murp_body,
        out_type=jax.ShapeDtypeStruct((B,), jnp.float32),
        mesh=mesh,
        scratch_types=[
            pltpu.VMEM((_BPW,), jnp.int32),
            pltpu.VMEM((_BPW,), jnp.int32),
            pltpu.VMEM((_BPW,), jnp.int32),
            pltpu.VMEM((_NCH, _CH), jnp.int32),
            pltpu.VMEM((_NCH, _CH), jnp.int32),
            pltpu.VMEM((_NCH, _CH), jnp.int32),
            pltpu.VMEM((_CH, PACK * DIM), jnp.float32),
            pltpu.VMEM((_CH, PACK * DIM), jnp.float32),
            pltpu.VMEM((_CH, DIM), jnp.float32),
            pltpu.VMEM((_CH, DIM), jnp.float32),
            pltpu.VMEM((_BPW,), jnp.float32),
            pltpu.VMEM((_BPW,), jnp.float32),
            pltpu.VMEM((_BPW,), jnp.float32),
            pltpu.SemaphoreType.DMA,
        ],
        compiler_params=pltpu.CompilerParams(
            needs_layout_passes=False, use_tc_tiling_on_sc=False),
    )
    eh2 = _pack_entities(jnp.swapaxes(Eh, 0, 1))
    return run(u_idx.astype(jnp.int32), r_idx.astype(jnp.int32),
               v_idx.astype(jnp.int32), eh2, rvh, Wu, bs, bo)
